# baseline (speedup 1.0000x reference)
import functools

import jax
import jax.numpy as jnp
from jax import lax
from jax.experimental import pallas as pl
from jax.experimental.pallas import tpu as pltpu
from jax.experimental.pallas import tpu_sc as plsc

_NUM_CORES = 2
_NUM_SUBCORES = 16
_NUM_WORKERS = _NUM_CORES * _NUM_SUBCORES


@functools.partial(jax.jit, static_argnames=("chunk", "nbuf"))
def _sc_gather(table, idx, chunk=8, nbuf=4):
    n = idx.shape[0]
    _, d = table.shape
    per_w = n // _NUM_WORKERS
    n_chunks = per_w // chunk
    n_steps = n_chunks // nbuf

    mesh = plsc.VectorSubcoreMesh(core_axis_name="c", subcore_axis_name="s")

    @functools.partial(
        pl.kernel,
        out_type=jax.ShapeDtypeStruct((n, d), jnp.float32),
        mesh=mesh,
        scratch_types=[
            pltpu.VMEM((per_w,), jnp.int32),
            pltpu.VMEM((nbuf, chunk, d), jnp.float32),
            pltpu.SemaphoreType.DMA((nbuf,)),
            pltpu.SemaphoreType.DMA((nbuf,)),
        ],
    )
    def body(table_hbm, idx_hbm, out_hbm, idx_v, rows_v, gsem, osem):
        wid = lax.axis_index("s") * _NUM_CORES + lax.axis_index("c")
        base = wid * per_w
        pltpu.sync_copy(idx_hbm.at[pl.ds(base, per_w)], idx_v)

        def gather(c, b):
            off = pl.multiple_of(c * chunk, 8)
            return pltpu.make_async_copy(
                table_hbm.at[idx_v.at[pl.ds(off, chunk)]],
                rows_v.at[b],
                gsem.at[b],
            )

        def writeout(c, b):
            off = pl.multiple_of(c * chunk, 8)
            return pltpu.make_async_copy(
                rows_v.at[b],
                out_hbm.at[pl.ds(base + off, chunk)],
                osem.at[b],
            )

        def step(t, carry):
            for b in range(nbuf):
                c = t * nbuf + b

                @pl.when(t > 0)
                def _():
                    writeout(c - nbuf, b).wait()

                gather(c, b).start()
            for b in range(nbuf):
                c = t * nbuf + b
                gather(c, b).wait()
                writeout(c, b).start()
            return carry

        lax.fori_loop(0, n_steps, step, 0)
        rem = n_chunks - n_steps * nbuf
        for b in range(rem):
            c = n_steps * nbuf + b
            writeout(c - nbuf, b).wait()
            gather(c, b).start()
        for b in range(rem):
            c = n_steps * nbuf + b
            gather(c, b).wait()
            writeout(c, b).start()
            writeout(c, b).wait()
        for b in range(rem, nbuf):
            writeout((n_steps - 1) * nbuf + b, b).wait()

    return body(table, idx)


def kernel(input_ids, embed_tokens):
    b, s = input_ids.shape
    _, d = embed_tokens.shape
    idx = input_ids.reshape(-1).astype(jnp.int32)
    out = _sc_gather(embed_tokens, idx)
    return out.reshape(b, s, d)


# speedup vs baseline: 1.0153x; 1.0153x over previous
import functools

import jax
import jax.numpy as jnp
from jax import lax
from jax.experimental import pallas as pl
from jax.experimental.pallas import tpu as pltpu
from jax.experimental.pallas import tpu_sc as plsc

_NUM_CORES = 2
_NUM_SUBCORES = 16
_NUM_WORKERS = _NUM_CORES * _NUM_SUBCORES


@functools.partial(jax.jit, static_argnames=("chunk", "nbuf"))
def _sc_gather(table, idx, chunk=8, nbuf=6):
    n = idx.shape[0]
    _, d = table.shape
    per_w = n // _NUM_WORKERS
    n_chunks = per_w // chunk
    n_steps = n_chunks // nbuf

    mesh = plsc.VectorSubcoreMesh(core_axis_name="c", subcore_axis_name="s")

    @functools.partial(
        pl.kernel,
        out_type=jax.ShapeDtypeStruct((n, d), jnp.float32),
        mesh=mesh,
        scratch_types=[
            pltpu.VMEM((per_w,), jnp.int32),
            pltpu.VMEM((nbuf, chunk, d), jnp.float32),
            pltpu.SemaphoreType.DMA((nbuf,)),
            pltpu.SemaphoreType.DMA((nbuf,)),
        ],
    )
    def body(table_hbm, idx_hbm, out_hbm, idx_v, rows_v, gsem, osem):
        wid = lax.axis_index("s") * _NUM_CORES + lax.axis_index("c")
        base = wid * per_w
        pltpu.sync_copy(idx_hbm.at[pl.ds(base, per_w)], idx_v)

        def gather(c, b):
            off = pl.multiple_of(c * chunk, 8)
            return pltpu.make_async_copy(
                table_hbm.at[idx_v.at[pl.ds(off, chunk)]],
                rows_v.at[b],
                gsem.at[b],
            )

        def writeout(c, b):
            off = pl.multiple_of(c * chunk, 8)
            return pltpu.make_async_copy(
                rows_v.at[b],
                out_hbm.at[pl.ds(base + off, chunk)],
                osem.at[b],
            )

        def step(t, carry):
            for b in range(nbuf):
                c = t * nbuf + b

                @pl.when(t > 0)
                def _():
                    writeout(c - nbuf, b).wait()

                gather(c, b).start()
            for b in range(nbuf):
                c = t * nbuf + b
                gather(c, b).wait()
                writeout(c, b).start()
            return carry

        lax.fori_loop(0, n_steps, step, 0)
        rem = n_chunks - n_steps * nbuf
        for b in range(rem):
            c = n_steps * nbuf + b
            writeout(c - nbuf, b).wait()
            gather(c, b).start()
        for b in range(rem):
            c = n_steps * nbuf + b
            gather(c, b).wait()
            writeout(c, b).start()
            writeout(c, b).wait()
        for b in range(rem, nbuf):
            writeout((n_steps - 1) * nbuf + b, b).wait()

    return body(table, idx)


def kernel(input_ids, embed_tokens):
    b, s = input_ids.shape
    _, d = embed_tokens.shape
    idx = input_ids.reshape(-1).astype(jnp.int32)
    out = _sc_gather(embed_tokens, idx)
    return out.reshape(b, s, d)
